# Initial kernel scaffold; baseline (speedup 1.0000x reference)
import jax
import jax.numpy as jnp
from jax.experimental import pallas as pl


def kernel(x, y, W_qkv, Wg, bg, Wp, bp, expert_bias):
    raise NotImplementedError("write your pallas kernel here")


# speedup vs baseline: 2.0053x; 2.0053x over previous
import functools

import jax
import jax.numpy as jnp
from jax.experimental import pallas as pl
from jax.experimental.pallas import tpu as pltpu

B = 2048
DIM = 1024
E = 8
H = 16
DH = 64
EPAD = 128
TBLK = 256
NEG = -1e30


def _qkv_body(x_ref, y_ref, wgt_ref, bg_ref, eb_ref, w_ref, out_ref, w_scr):
    e = pl.program_id(1)

    @pl.when(e == 0)
    def _gate():
        logits = jnp.dot(x_ref[...], wgt_ref[...],
                         preferred_element_type=jnp.float32) + bg_ref[0]
        m = jnp.max(logits, axis=-1, keepdims=True)
        p = jnp.exp(logits - m)
        probs = p / jnp.sum(p, axis=-1, keepdims=True)
        scores = probs + eb_ref[0]
        lane = jax.lax.broadcasted_iota(jnp.int32, (TBLK, EPAD), 1)
        v1 = jnp.max(scores, axis=-1, keepdims=True)
        i1 = jnp.min(jnp.where(scores == v1, lane, EPAD), axis=-1, keepdims=True)
        s2 = jnp.where(lane == i1, NEG, scores)
        v2 = jnp.max(s2, axis=-1, keepdims=True)
        i2 = jnp.min(jnp.where(s2 == v2, lane, EPAD), axis=-1, keepdims=True)
        w_scr[...] = (jnp.where(lane == i1, v1, 0.0)
                      + jnp.where(lane == i2, v2, 0.0))
        out_ref[...] = jnp.zeros_like(out_ref)

    lane = jax.lax.broadcasted_iota(jnp.int32, (TBLK, EPAD), 1)
    we = jnp.sum(jnp.where(lane == e, w_scr[...], 0.0), axis=-1, keepdims=True)
    w3 = w_ref[0]
    yw = y_ref[...] * we
    xw = x_ref[...] * we
    out_ref[:, :DIM] += jnp.dot(yw, w3[:, :DIM],
                                preferred_element_type=jnp.float32)
    out_ref[:, DIM:] += jnp.dot(xw, w3[:, DIM:],
                                preferred_element_type=jnp.float32)


def _attn_body(q_ref, k_ref, v_ref, amask_ref, o_ref):
    scale = DH ** -0.5
    for i in range(8):
        sl = slice(i * 128, (i + 1) * 128)
        qb = q_ref[sl, :]
        kb = k_ref[sl, :]
        vb = v_ref[sl, :]
        s = jax.lax.dot_general(qb, kb, (((1,), (1,)), ((), ())),
                                preferred_element_type=jnp.float32)
        s = s * scale + amask_ref[...]
        m = jnp.max(s, axis=-1, keepdims=True)
        p = jnp.exp(s - m)
        attn = p / jnp.sum(p, axis=-1, keepdims=True)
        o_ref[sl, :] = jax.lax.dot_general(
            attn, vb, (((1,), (0,)), ((), ())),
            preferred_element_type=jnp.float32)


def _proj_body(o_ref, wp_ref, bp_ref, out_ref):
    out_ref[...] = jnp.dot(o_ref[...], wp_ref[...],
                           preferred_element_type=jnp.float32) + bp_ref[0]


def _attention_and_proj(qkv, Wp, bp):
    q_r = qkv[:, :DIM].reshape(B * H, DH)
    k_r = qkv[:, DIM:2 * DIM].reshape(B * H, DH)
    v_r = qkv[:, 2 * DIM:].reshape(B * H, DH)
    r = jax.lax.broadcasted_iota(jnp.int32, (128, 128), 0) // H
    c = jax.lax.broadcasted_iota(jnp.int32, (128, 128), 1) // H
    amask = jnp.where(r == c, 0.0, NEG).astype(jnp.float32)

    RB = 1024
    o_r = pl.pallas_call(
        _attn_body,
        grid=(B * H // RB,),
        in_specs=[
            pl.BlockSpec((RB, DH), lambda i: (i, 0)),
            pl.BlockSpec((RB, DH), lambda i: (i, 0)),
            pl.BlockSpec((RB, DH), lambda i: (i, 0)),
            pl.BlockSpec((128, 128), lambda i: (0, 0)),
        ],
        out_specs=pl.BlockSpec((RB, DH), lambda i: (i, 0)),
        out_shape=jax.ShapeDtypeStruct((B * H, DH), jnp.float32),
    )(q_r, k_r, v_r, amask)

    o_flat = o_r.reshape(B, DIM)
    wp_perm = Wp.T.reshape(DH, H, DIM).transpose(1, 0, 2).reshape(DIM, DIM)
    return pl.pallas_call(
        _proj_body,
        grid=(B // TBLK,),
        in_specs=[
            pl.BlockSpec((TBLK, DIM), lambda i: (i, 0)),
            pl.BlockSpec((DIM, DIM), lambda i: (0, 0)),
            pl.BlockSpec((1, DIM), lambda i: (0, 0)),
        ],
        out_specs=pl.BlockSpec((TBLK, DIM), lambda i: (i, 0)),
        out_shape=jax.ShapeDtypeStruct((B, DIM), jnp.float32),
    )(o_flat, wp_perm, bp[None, :])


def kernel(x, y, W_qkv, Wg, bg, Wp, bp, expert_bias):
    wgt = jnp.pad(Wg.T, ((0, 0), (0, EPAD - E)))
    bgp = jnp.pad(bg, (0, EPAD - E), constant_values=NEG)[None, :]
    ebp = jnp.pad(expert_bias, (0, EPAD - E), constant_values=NEG)[None, :]

    qkv = pl.pallas_call(
        _qkv_body,
        grid=(B // TBLK, E),
        in_specs=[
            pl.BlockSpec((TBLK, DIM), lambda t, e: (t, 0)),
            pl.BlockSpec((TBLK, DIM), lambda t, e: (t, 0)),
            pl.BlockSpec((DIM, EPAD), lambda t, e: (0, 0)),
            pl.BlockSpec((1, EPAD), lambda t, e: (0, 0)),
            pl.BlockSpec((1, EPAD), lambda t, e: (0, 0)),
            pl.BlockSpec((1, DIM, 3 * DIM), lambda t, e: (e, 0, 0)),
        ],
        out_specs=pl.BlockSpec((TBLK, 3 * DIM), lambda t, e: (t, 0)),
        out_shape=jax.ShapeDtypeStruct((B, 3 * DIM), jnp.float32),
        scratch_shapes=[pltpu.VMEM((TBLK, EPAD), jnp.float32)],
        compiler_params=pltpu.CompilerParams(
            dimension_semantics=("arbitrary", "arbitrary")),
    )(x, y, wgt, bgp, ebp, W_qkv)

    return _attention_and_proj(qkv, Wp, bp)


# speedup vs baseline: 2.1879x; 1.0910x over previous
import functools

import jax
import jax.numpy as jnp
from jax.experimental import pallas as pl
from jax.experimental.pallas import tpu as pltpu

B = 2048
DIM = 1024
E = 8
H = 16
DH = 64
EPAD = 128
TBLK = 256
NEG = -1e30


def _qkv_body(x_ref, y_ref, wgt_ref, bg_ref, eb_ref, w_ref, out_ref, w_scr):
    e = pl.program_id(1)

    @pl.when(e == 0)
    def _gate():
        logits = jnp.dot(x_ref[...], wgt_ref[...],
                         preferred_element_type=jnp.float32) + bg_ref[0]
        m = jnp.max(logits, axis=-1, keepdims=True)
        p = jnp.exp(logits - m)
        probs = p / jnp.sum(p, axis=-1, keepdims=True)
        scores = probs + eb_ref[0]
        lane = jax.lax.broadcasted_iota(jnp.int32, (TBLK, EPAD), 1)
        v1 = jnp.max(scores, axis=-1, keepdims=True)
        i1 = jnp.min(jnp.where(scores == v1, lane, EPAD), axis=-1, keepdims=True)
        s2 = jnp.where(lane == i1, NEG, scores)
        v2 = jnp.max(s2, axis=-1, keepdims=True)
        i2 = jnp.min(jnp.where(s2 == v2, lane, EPAD), axis=-1, keepdims=True)
        w_scr[...] = (jnp.where(lane == i1, v1, 0.0)
                      + jnp.where(lane == i2, v2, 0.0))
        out_ref[...] = jnp.zeros_like(out_ref)

    lane = jax.lax.broadcasted_iota(jnp.int32, (TBLK, EPAD), 1)
    we = jnp.sum(jnp.where(lane == e, w_scr[...], 0.0), axis=-1, keepdims=True)
    w3 = w_ref[0]
    yw = (y_ref[...] * we).astype(jnp.bfloat16)
    xw = (x_ref[...] * we).astype(jnp.bfloat16)
    out_ref[:, :DIM] += jnp.dot(yw, w3[:, :DIM],
                                preferred_element_type=jnp.float32)
    out_ref[:, DIM:] += jnp.dot(xw, w3[:, DIM:],
                                preferred_element_type=jnp.float32)


def _attn_body(q_ref, k_ref, v_ref, amask_ref, o_ref):
    scale = DH ** -0.5
    for i in range(8):
        sl = slice(i * 128, (i + 1) * 128)
        qb = q_ref[sl, :]
        kb = k_ref[sl, :]
        vb = v_ref[sl, :]
        s = jax.lax.dot_general(qb, kb, (((1,), (1,)), ((), ())),
                                preferred_element_type=jnp.float32)
        s = s * scale + amask_ref[...]
        m = jnp.max(s, axis=-1, keepdims=True)
        p = jnp.exp(s - m)
        attn = p / jnp.sum(p, axis=-1, keepdims=True)
        o_ref[sl, :] = jax.lax.dot_general(
            attn, vb, (((1,), (0,)), ((), ())),
            preferred_element_type=jnp.float32)


def _proj_body(o_ref, wp_ref, bp_ref, out_ref):
    out_ref[...] = jnp.dot(o_ref[...], wp_ref[...],
                           preferred_element_type=jnp.float32) + bp_ref[0]


def _attention_and_proj(qkv, Wp, bp):
    q_r = qkv[:, :DIM].reshape(B * H, DH)
    k_r = qkv[:, DIM:2 * DIM].reshape(B * H, DH)
    v_r = qkv[:, 2 * DIM:].reshape(B * H, DH)
    r = jax.lax.broadcasted_iota(jnp.int32, (128, 128), 0) // H
    c = jax.lax.broadcasted_iota(jnp.int32, (128, 128), 1) // H
    amask = jnp.where(r == c, 0.0, NEG).astype(jnp.float32)

    RB = 1024
    o_r = pl.pallas_call(
        _attn_body,
        grid=(B * H // RB,),
        in_specs=[
            pl.BlockSpec((RB, DH), lambda i: (i, 0)),
            pl.BlockSpec((RB, DH), lambda i: (i, 0)),
            pl.BlockSpec((RB, DH), lambda i: (i, 0)),
            pl.BlockSpec((128, 128), lambda i: (0, 0)),
        ],
        out_specs=pl.BlockSpec((RB, DH), lambda i: (i, 0)),
        out_shape=jax.ShapeDtypeStruct((B * H, DH), jnp.float32),
    )(q_r, k_r, v_r, amask)

    o_flat = o_r.reshape(B, DIM)
    wp_perm = Wp.T.reshape(DH, H, DIM).transpose(1, 0, 2).reshape(DIM, DIM)
    return pl.pallas_call(
        _proj_body,
        grid=(B // TBLK,),
        in_specs=[
            pl.BlockSpec((TBLK, DIM), lambda i: (i, 0)),
            pl.BlockSpec((DIM, DIM), lambda i: (0, 0)),
            pl.BlockSpec((1, DIM), lambda i: (0, 0)),
        ],
        out_specs=pl.BlockSpec((TBLK, DIM), lambda i: (i, 0)),
        out_shape=jax.ShapeDtypeStruct((B, DIM), jnp.float32),
    )(o_flat, wp_perm, bp[None, :])


def kernel(x, y, W_qkv, Wg, bg, Wp, bp, expert_bias):
    wgt = jnp.pad(Wg.T, ((0, 0), (0, EPAD - E)))
    bgp = jnp.pad(bg, (0, EPAD - E), constant_values=NEG)[None, :]
    ebp = jnp.pad(expert_bias, (0, EPAD - E), constant_values=NEG)[None, :]

    qkv = pl.pallas_call(
        _qkv_body,
        grid=(B // TBLK, E),
        in_specs=[
            pl.BlockSpec((TBLK, DIM), lambda t, e: (t, 0)),
            pl.BlockSpec((TBLK, DIM), lambda t, e: (t, 0)),
            pl.BlockSpec((DIM, EPAD), lambda t, e: (0, 0)),
            pl.BlockSpec((1, EPAD), lambda t, e: (0, 0)),
            pl.BlockSpec((1, EPAD), lambda t, e: (0, 0)),
            pl.BlockSpec((1, DIM, 3 * DIM), lambda t, e: (e, 0, 0)),
        ],
        out_specs=pl.BlockSpec((TBLK, 3 * DIM), lambda t, e: (t, 0)),
        out_shape=jax.ShapeDtypeStruct((B, 3 * DIM), jnp.float32),
        scratch_shapes=[pltpu.VMEM((TBLK, EPAD), jnp.float32)],
        compiler_params=pltpu.CompilerParams(
            dimension_semantics=("arbitrary", "arbitrary")),
    )(x, y, wgt, bgp, ebp, W_qkv.astype(jnp.bfloat16))

    return _attention_and_proj(qkv, Wp, bp)
